# Initial kernel scaffold; baseline (speedup 1.0000x reference)
import jax
import jax.numpy as jnp
from jax.experimental import pallas as pl


def kernel(ids, coords, word, pos0, pos1, pos2, pos3):
    raise NotImplementedError("write your pallas kernel here")


# speedup vs baseline: 1.6104x; 1.6104x over previous
import functools

import jax
import jax.numpy as jnp
from jax import lax
from jax.experimental import pallas as pl
from jax.experimental.pallas import tpu as pltpu
from jax.experimental.pallas import tpu_sc as plsc

NC = 2
NS = 16
L = 16
NW = NC * NS
N_TOK = 4 * 2048
D = 768
TOK_PER_W = N_TOK // NW
T = 64
NCHUNK = TOK_PER_W // T
DV = D // L

_mesh = plsc.VectorSubcoreMesh(core_axis_name="c", subcore_axis_name="s")


@functools.partial(
    pl.kernel,
    out_type=jax.ShapeDtypeStruct((N_TOK, D), jnp.float32),
    mesh=_mesh,
    scratch_types=[
        pltpu.VMEM((T,), jnp.int32),
        pltpu.VMEM((T, D), jnp.float32),
        pltpu.VMEM((T, D), jnp.float32),
        pltpu.SemaphoreType.DMA,
    ],
)
def _embed4(ids_hbm, c0_hbm, c1_hbm, c2_hbm, c3_hbm,
            word_hbm, p0_hbm, p1_hbm, p2_hbm, p3_hbm,
            out_hbm, idx_v, acc_v, tmp_v, sem):
    wid = lax.axis_index("s") * NC + lax.axis_index("c")
    wbase = wid * TOK_PER_W

    def chunk_body(i, carry):
        base = wbase + i * T
        pltpu.sync_copy(ids_hbm.at[pl.ds(base, T)], idx_v)
        pltpu.async_copy(word_hbm.at[idx_v], acc_v, sem).wait()
        for c_hbm, p_hbm in ((c0_hbm, p0_hbm), (c1_hbm, p1_hbm),
                             (c2_hbm, p2_hbm), (c3_hbm, p3_hbm)):
            pltpu.sync_copy(c_hbm.at[pl.ds(base, T)], idx_v)
            pltpu.async_copy(p_hbm.at[idx_v], tmp_v, sem).wait()

            def add_row(t, c2_):
                for j in range(DV):
                    sl = pl.ds(j * L, L)
                    plsc.addupdate(acc_v.at[t, sl], tmp_v[t, sl])
                return c2_

            lax.fori_loop(0, T, add_row, 0)
        pltpu.sync_copy(acc_v, out_hbm.at[pl.ds(base, T)])
        return carry

    lax.fori_loop(0, NCHUNK, chunk_body, 0)


def kernel(ids, coords, word, pos0, pos1, pos2, pos3):
    B, S = ids.shape
    ids_f = ids.reshape(N_TOK).astype(jnp.int32)
    c = coords.reshape(N_TOK, 4).astype(jnp.int32)
    out = _embed4(ids_f, c[:, 0], c[:, 1], c[:, 2], c[:, 3],
                  word, pos0, pos1, pos2, pos3)
    return out.reshape(B, S, D)


# speedup vs baseline: 1.9013x; 1.1807x over previous
import functools

import jax
import jax.numpy as jnp
from jax import lax
from jax.experimental import pallas as pl
from jax.experimental.pallas import tpu as pltpu
from jax.experimental.pallas import tpu_sc as plsc

NC = 2
NS = 16
L = 16
NW = NC * NS
N_TOK = 4 * 2048
D = 768
TOK_PER_W = N_TOK // NW
T = 32
NCHUNK = TOK_PER_W // T
DV = D // L

_mesh = plsc.VectorSubcoreMesh(core_axis_name="c", subcore_axis_name="s")


@functools.partial(
    pl.kernel,
    out_type=jax.ShapeDtypeStruct((N_TOK, D), jnp.float32),
    mesh=_mesh,
    scratch_types=[
        pltpu.VMEM((TOK_PER_W,), jnp.int32),
        pltpu.VMEM((TOK_PER_W,), jnp.int32),
        pltpu.VMEM((TOK_PER_W,), jnp.int32),
        pltpu.VMEM((TOK_PER_W,), jnp.int32),
        pltpu.VMEM((TOK_PER_W,), jnp.int32),
        pltpu.VMEM((T, D), jnp.float32),
        pltpu.VMEM((T, D), jnp.float32),
        pltpu.VMEM((T, D), jnp.float32),
        pltpu.VMEM((T, D), jnp.float32),
        pltpu.SemaphoreType.DMA,
        pltpu.SemaphoreType.DMA,
        pltpu.SemaphoreType.DMA,
        pltpu.SemaphoreType.DMA,
        pltpu.SemaphoreType.DMA,
    ],
)
def _embed4(ids_hbm, c0_hbm, c1_hbm, c2_hbm, c3_hbm,
            word_hbm, p0_hbm, p1_hbm, p2_hbm, p3_hbm,
            out_hbm, idsb, c0b, c1b, c2b, c3b,
            acc0, acc1, tmp0, tmp1,
            sem_w, sem_t0, sem_t1, sem_o0, sem_o1):
    wid = lax.axis_index("s") * NC + lax.axis_index("c")
    wbase = wid * TOK_PER_W

    pltpu.sync_copy(ids_hbm.at[pl.ds(wbase, TOK_PER_W)], idsb)
    pltpu.sync_copy(c0_hbm.at[pl.ds(wbase, TOK_PER_W)], c0b)
    pltpu.sync_copy(c1_hbm.at[pl.ds(wbase, TOK_PER_W)], c1b)
    pltpu.sync_copy(c2_hbm.at[pl.ds(wbase, TOK_PER_W)], c2b)
    pltpu.sync_copy(c3_hbm.at[pl.ds(wbase, TOK_PER_W)], c3b)

    accs = (acc0, acc1)
    sems_o = (sem_o0, sem_o1)

    def add_pass(accr, tmpr):
        def row(t, c):
            for j in range(DV):
                sl = pl.ds(j * L, L)
                plsc.addupdate(accr.at[t, sl], tmpr[t, sl])
            return c
        lax.fori_loop(0, T, row, 0)

    def chunk_body(i, a, first):
        off = i * T
        gbase = wbase + off
        acc = accs[a]
        out_dst = out_hbm.at[pl.ds(gbase, T)]
        if not first:
            pltpu.make_async_copy(acc, out_dst, sems_o[a]).wait()
        w = pltpu.async_copy(word_hbm.at[idsb.at[pl.ds(off, T)]], acc, sem_w)
        g0 = pltpu.async_copy(p0_hbm.at[c0b.at[pl.ds(off, T)]], tmp0, sem_t0)
        g1 = pltpu.async_copy(p1_hbm.at[c1b.at[pl.ds(off, T)]], tmp1, sem_t1)
        w.wait()
        g0.wait()
        add_pass(acc, tmp0)
        g2 = pltpu.async_copy(p2_hbm.at[c2b.at[pl.ds(off, T)]], tmp0, sem_t0)
        g1.wait()
        add_pass(acc, tmp1)
        g3 = pltpu.async_copy(p3_hbm.at[c3b.at[pl.ds(off, T)]], tmp1, sem_t1)
        g2.wait()
        add_pass(acc, tmp0)
        g3.wait()
        add_pass(acc, tmp1)
        pltpu.async_copy(acc, out_dst, sems_o[a])

    chunk_body(0, 0, True)
    chunk_body(1, 1, True)

    def loop_body(k, c):
        chunk_body(2 * k, 0, False)
        chunk_body(2 * k + 1, 1, False)
        return c

    lax.fori_loop(1, NCHUNK // 2, loop_body, 0)

    tail = wbase + (NCHUNK - 2) * T
    pltpu.make_async_copy(acc0, out_hbm.at[pl.ds(tail, T)], sem_o0).wait()
    pltpu.make_async_copy(acc1, out_hbm.at[pl.ds(tail + T, T)], sem_o1).wait()


def kernel(ids, coords, word, pos0, pos1, pos2, pos3):
    B, S = ids.shape
    ids_f = ids.reshape(N_TOK).astype(jnp.int32)
    c = coords.reshape(N_TOK, 4).astype(jnp.int32)
    out = _embed4(ids_f, c[:, 0], c[:, 1], c[:, 2], c[:, 3],
                  word, pos0, pos1, pos2, pos3)
    return out.reshape(B, S, D)
